# baseline (speedup 1.0000x reference)
import functools

import jax
import jax.numpy as jnp
from jax import lax
from jax.experimental import pallas as pl
from jax.experimental.pallas import tpu as pltpu
from jax.experimental.pallas import tpu_sc as plsc

VOCAB = 100000
D = 64
NSTATE = 4
B = 4096
S = 200
TOKENS = B * S

NC = 2
NSUB = 16
L = 16
NW = NC * NSUB
TPW = TOKENS // NW
CH = 128
NCHUNK = TPW // CH

_F32 = jnp.float32
_I32 = jnp.int32


def _rsqrt(x):
    i = plsc.bitcast(x, _I32)
    i = jnp.int32(0x5F3759DF) - lax.shift_right_logical(i, 1)
    y = plsc.bitcast(i, _F32)
    for _ in range(2):
        y = y * (1.5 - 0.5 * x * y * y)
    return y


def _body(ids, table, sm, pos, tte, gamma, beta, out_emb, out_unc,
          padd_v, sm_v, tte_v, gam_v, bet_v,
          idx0_v, idx1_v, rows0_v, rows1_v, emb0_v, emb1_v, sbuf_v,
          unc0_v, unc1_v, tbuf_v, t2buf_v, mu_v, rstd_v,
          gsem0, gsem1, isem0, isem1, osem0, osem1, usem0, usem1):
    cid = lax.axis_index("c")
    sid = lax.axis_index("s")
    wid = sid * NC + cid
    t0 = wid * TPW

    pltpu.sync_copy(pos, padd_v)
    pltpu.sync_copy(sm, sm_v)
    pltpu.sync_copy(tte, tte_v)
    pltpu.sync_copy(gamma, gam_v)
    pltpu.sync_copy(beta, bet_v)

    iota = lax.iota(_I32, L)
    lm = lax.rem(iota, 4)
    smv = sm_v[...]
    zeros = jnp.zeros((L,), _F32)
    w = [jnp.full((L,), jnp.sum(jnp.where(lm == j, smv, zeros)) * 0.25, _F32)
         for j in range(4)]
    ttek = [tte_v[pl.ds(L * k, L)] for k in range(4)]
    gk = [gam_v[pl.ds(L * k, L)] for k in range(4)]
    bk = [bet_v[pl.ds(L * k, L)] for k in range(4)]
    lane0 = iota == 0
    io3 = lax.shift_right_logical(iota, 3)
    io7 = lax.bitwise_and(iota, 7)

    def prow(s_, carry):
        for k in range(4):
            o = s_ * D + k * L
            padd_v[pl.ds(o, L)] = padd_v[pl.ds(o, L)] + ttek[k]
        return carry

    lax.fori_loop(0, S, prow, 0)

    idx_b = (idx0_v, idx1_v)
    rows_b = (rows0_v, rows1_v)
    emb_b = (emb0_v, emb1_v)
    unc_b = (unc0_v, unc1_v)
    gsem_b = (gsem0, gsem1)
    isem_b = (isem0, isem1)
    osem_b = (osem0, osem1)
    usem_b = (usem0, usem1)

    for b in (0, 1):
        pltpu.sync_copy(ids.at[pl.ds(t0 + b * CH, CH)], idx_b[b])
        pltpu.async_copy(table.at[idx_b[b]], rows_b[b], gsem_b[b])

    def out_slices(c):
        u0 = t0 + c * CH
        s_ = lax.div(u0, B)
        bidx = lax.div(lax.rem(u0, B), CH)
        return (out_emb.at[s_, :, bidx, :, :], out_unc.at[s_, pl.ds(
            lax.rem(u0, B), CH)])

    def run_chunk(b, c):
        u0 = t0 + c * CH
        rows_v = rows_b[b]
        emb_v = emb_b[b]
        unc_v = unc_b[b]
        pltpu.make_async_copy(table.at[idx_b[b]], rows_v, gsem_b[b]).wait()

        @pl.when(c + 2 < NCHUNK)
        def _():
            pltpu.async_copy(ids.at[pl.ds(u0 + 2 * CH, CH)], idx_b[b],
                             isem_b[b])

        @pl.when(c >= 2)
        def _():
            oe, ou = out_slices(c - 2)
            pltpu.make_async_copy(emb_v, oe, osem_b[b]).wait()
            pltpu.make_async_copy(unc_v, ou, usem_b[b]).wait()

        s_ = lax.div(u0, B)
        pv = [padd_v[pl.ds(s_ * D + k * L, L)] for k in range(4)]

        @plsc.parallel_loop(0, CH, unroll=8)
        def tok(i):
            T = zeros
            T2 = zeros
            U = zeros
            ifull = jnp.full((L,), i, _I32)
            for k in range(4):
                xs = [rows_v[i, pl.ds(j * D + k * L, L)] for j in range(4)]
                word = xs[0] * w[0] + xs[1] * w[1] + xs[2] * w[2] + xs[3] * w[3]
                sm4 = xs[0] + xs[1] + xs[2] + xs[3]
                ss4 = xs[0] * xs[0] + xs[1] * xs[1] + xs[2] * xs[2] + xs[3] * xs[3]
                var = ss4 * (1.0 / 3.0) - sm4 * sm4 * (1.0 / 12.0)
                U = U + var * var
                e = word + pv[k]
                plsc.store_scatter(emb_v, [io3 + 2 * k, io7, ifull], e)
                T = T + e
                T2 = T2 + e * e
            plsc.store_scatter(tbuf_v, [ifull],
                               jnp.full((L,), jnp.sum(T), _F32), mask=lane0)
            plsc.store_scatter(t2buf_v, [ifull],
                               jnp.full((L,), jnp.sum(T2), _F32), mask=lane0)
            plsc.store_scatter(sbuf_v, [ifull],
                               jnp.full((L,), jnp.sum(U), _F32), mask=lane0)

        @plsc.parallel_loop(0, CH // L, unroll=2)
        def stat_pass(g):
            Tv = tbuf_v[pl.ds(g * L, L)]
            T2v = t2buf_v[pl.ds(g * L, L)]
            mu = Tv * (1.0 / 64.0)
            varln = jnp.maximum(T2v * (1.0 / 64.0) - mu * mu, 0.0) + 1e-12
            mu_v[pl.ds(g * L, L)] = mu
            rstd_v[pl.ds(g * L, L)] = _rsqrt(varln)
            v = sbuf_v[pl.ds(g * L, L)]
            unc_v[pl.ds(g * L, L)] = v * _rsqrt(jnp.maximum(v, 1e-30))

        @plsc.parallel_loop(0, CH, unroll=8)
        def tok_norm(i):
            ifull = jnp.full((L,), i, _I32)
            musp = plsc.load_gather(mu_v, [ifull])
            rssp = plsc.load_gather(rstd_v, [ifull])
            for k in range(4):
                d8 = io3 + 2 * k
                e = plsc.load_gather(emb_v, [d8, io7, ifull])
                plsc.store_scatter(emb_v, [d8, io7, ifull],
                                   (e - musp) * (rssp * gk[k]) + bk[k])

        oe, ou = out_slices(c)
        pltpu.async_copy(emb_v, oe, osem_b[b])
        pltpu.async_copy(unc_v, ou, usem_b[b])

        @pl.when(c + 2 < NCHUNK)
        def _():
            pltpu.make_async_copy(ids.at[pl.ds(u0 + 2 * CH, CH)], idx_b[b],
                                  isem_b[b]).wait()
            pltpu.async_copy(table.at[idx_b[b]], rows_b[b], gsem_b[b])

    def pair_body(p, carry):
        for b in (0, 1):
            run_chunk(b, 2 * p + b)
        return carry

    lax.fori_loop(0, NCHUNK // 2, pair_body, 0)

    for b in (0, 1):
        oe, ou = out_slices(NCHUNK - 2 + b)
        pltpu.make_async_copy(emb_b[b], oe, osem_b[b]).wait()
        pltpu.make_async_copy(unc_b[b], ou, usem_b[b]).wait()


@jax.jit
def _run(ids, table, sm_flat, pos_flat, tte0, ln_gamma, ln_beta):
    mesh = plsc.VectorSubcoreMesh(core_axis_name="c", subcore_axis_name="s",
                                  num_cores=NC, num_subcores=NSUB)
    kern = pl.kernel(
        _body,
        out_type=[
            jax.ShapeDtypeStruct((S, D // 8, B // CH, 8, CH), _F32),
            jax.ShapeDtypeStruct((S, B), _F32),
        ],
        mesh=mesh,
        compiler_params=pltpu.CompilerParams(needs_layout_passes=False),
        scratch_types=[
            pltpu.VMEM((S * D,), _F32),
            pltpu.VMEM((16,), _F32),
            pltpu.VMEM((D,), _F32),
            pltpu.VMEM((D,), _F32),
            pltpu.VMEM((D,), _F32),
            pltpu.VMEM((CH,), _I32),
            pltpu.VMEM((CH,), _I32),
            pltpu.VMEM((CH, NSTATE * D), _F32),
            pltpu.VMEM((CH, NSTATE * D), _F32),
            pltpu.VMEM((D // 8, 8, CH), _F32),
            pltpu.VMEM((D // 8, 8, CH), _F32),
            pltpu.VMEM((CH,), _F32),
            pltpu.VMEM((CH,), _F32),
            pltpu.VMEM((CH,), _F32),
            pltpu.VMEM((CH,), _F32),
            pltpu.VMEM((CH,), _F32),
            pltpu.VMEM((CH,), _F32),
            pltpu.VMEM((CH,), _F32),
        ] + [pltpu.SemaphoreType.DMA] * 8,
    )
    return kern(ids, table, sm_flat, pos_flat, tte0, ln_gamma, ln_beta)


def kernel(input_ids, state_embeddings, superposition_matrix,
           position_embeddings, token_type_embeddings, ln_gamma, ln_beta):
    ids = input_ids.T.reshape(-1)
    table = state_embeddings.reshape(VOCAB, NSTATE * D)
    sm_flat = superposition_matrix.reshape(-1)
    pos_flat = position_embeddings[:S].reshape(-1)
    tte0 = token_type_embeddings[0]
    emb5, unc_t = _run(ids, table, sm_flat, pos_flat, tte0, ln_gamma, ln_beta)
    emb = emb5.transpose(2, 4, 0, 1, 3).reshape(B, S, D)
    return emb, unc_t.T


# speedup vs baseline: 2.5225x; 2.5225x over previous
import functools

import jax
import jax.numpy as jnp
from jax import lax
from jax.experimental import pallas as pl
from jax.experimental.pallas import tpu as pltpu
from jax.experimental.pallas import tpu_sc as plsc

VOCAB = 100000
D = 64
NSTATE = 4
B = 4096
S = 200
TOKENS = B * S

NC = 2
NSUB = 16
L = 16
NW = NC * NSUB
TPW = TOKENS // NW
CH = 128
NCHUNK = TPW // CH

_F32 = jnp.float32
_I32 = jnp.int32


def _rsqrt(x):
    i = plsc.bitcast(x, _I32)
    i = jnp.int32(0x5F3759DF) - lax.shift_right_logical(i, 1)
    y = plsc.bitcast(i, _F32)
    for _ in range(2):
        y = y * (1.5 - 0.5 * x * y * y)
    return y


def _body(ids, table, sm, pos, tte, gamma, beta, out_emb, out_unc,
          padd_v, sm_v, tte_v, gam_v, bet_v,
          idx0_v, idx1_v, rows0_v, rows1_v, emb0_v, emb1_v, sbuf_v,
          unc0_v, unc1_v, tbuf_v, t2buf_v, mu_v, rstd_v,
          gsem0, gsem1, isem0, isem1, osem0, osem1, usem0, usem1):
    cid = lax.axis_index("c")
    sid = lax.axis_index("s")
    wid = sid * NC + cid
    t0 = wid * TPW

    pltpu.sync_copy(pos, padd_v)
    pltpu.sync_copy(sm, sm_v)
    pltpu.sync_copy(tte, tte_v)
    pltpu.sync_copy(gamma, gam_v)
    pltpu.sync_copy(beta, bet_v)

    iota = lax.iota(_I32, L)
    lm = lax.rem(iota, 4)
    smv = sm_v[...]
    zeros = jnp.zeros((L,), _F32)
    w = [jnp.full((L,), jnp.sum(jnp.where(lm == j, smv, zeros)) * 0.25, _F32)
         for j in range(4)]
    ttek = [tte_v[pl.ds(L * k, L)] for k in range(4)]
    gk = [gam_v[pl.ds(L * k, L)] for k in range(4)]
    bk = [bet_v[pl.ds(L * k, L)] for k in range(4)]
    lane0 = iota == 0

    def prow(s_, carry):
        for k in range(4):
            o = s_ * D + k * L
            padd_v[pl.ds(o, L)] = padd_v[pl.ds(o, L)] + ttek[k]
        return carry

    lax.fori_loop(0, S, prow, 0)

    idx_b = (idx0_v, idx1_v)
    rows_b = (rows0_v, rows1_v)
    emb_b = (emb0_v, emb1_v)
    unc_b = (unc0_v, unc1_v)
    gsem_b = (gsem0, gsem1)
    isem_b = (isem0, isem1)
    osem_b = (osem0, osem1)
    usem_b = (usem0, usem1)

    for b in (0, 1):
        pltpu.sync_copy(ids.at[pl.ds(t0 + b * CH, CH)], idx_b[b])
        pltpu.async_copy(table.at[idx_b[b]], rows_b[b], gsem_b[b])

    def run_chunk(b, c):
        tb = t0 + c * CH
        rows_v = rows_b[b]
        emb_v = emb_b[b]
        unc_v = unc_b[b]
        pltpu.make_async_copy(table.at[idx_b[b]], rows_v, gsem_b[b]).wait()

        @pl.when(c + 2 < NCHUNK)
        def _():
            pltpu.async_copy(ids.at[pl.ds(tb + 2 * CH, CH)], idx_b[b],
                             isem_b[b])

        @pl.when(c >= 2)
        def _():
            pltpu.make_async_copy(
                emb_v, out_emb.at[pl.ds(tb - 2 * CH, CH)], osem_b[b]).wait()
            pltpu.make_async_copy(
                unc_v, out_unc.at[pl.ds(tb - 2 * CH, CH)], usem_b[b]).wait()

        @plsc.parallel_loop(0, CH, unroll=8)
        def tok(i):
            s = lax.rem(tb + i, S)
            T = zeros
            T2 = zeros
            U = zeros
            for k in range(4):
                xs = [rows_v[i, pl.ds(j * D + k * L, L)] for j in range(4)]
                word = xs[0] * w[0] + xs[1] * w[1] + xs[2] * w[2] + xs[3] * w[3]
                sm4 = xs[0] + xs[1] + xs[2] + xs[3]
                ss4 = xs[0] * xs[0] + xs[1] * xs[1] + xs[2] * xs[2] + xs[3] * xs[3]
                var = ss4 * (1.0 / 3.0) - sm4 * sm4 * (1.0 / 12.0)
                U = U + var * var
                e = word + padd_v[pl.ds(s * D + k * L, L)]
                emb_v[i, pl.ds(k * L, L)] = e
                T = T + e
                T2 = T2 + e * e
            ifull = jnp.full((L,), i, _I32)
            plsc.store_scatter(tbuf_v, [ifull],
                               jnp.full((L,), jnp.sum(T), _F32), mask=lane0)
            plsc.store_scatter(t2buf_v, [ifull],
                               jnp.full((L,), jnp.sum(T2), _F32), mask=lane0)
            plsc.store_scatter(sbuf_v, [ifull],
                               jnp.full((L,), jnp.sum(U), _F32), mask=lane0)

        @plsc.parallel_loop(0, CH // L, unroll=2)
        def stat_pass(g):
            Tv = tbuf_v[pl.ds(g * L, L)]
            T2v = t2buf_v[pl.ds(g * L, L)]
            mu = Tv * (1.0 / 64.0)
            varln = jnp.maximum(T2v * (1.0 / 64.0) - mu * mu, 0.0) + 1e-12
            mu_v[pl.ds(g * L, L)] = mu
            rstd_v[pl.ds(g * L, L)] = _rsqrt(varln)
            v = sbuf_v[pl.ds(g * L, L)]
            unc_v[pl.ds(g * L, L)] = v * _rsqrt(jnp.maximum(v, 1e-30))

        @plsc.parallel_loop(0, CH, unroll=8)
        def tok_norm(i):
            ifull = jnp.full((L,), i, _I32)
            musp = plsc.load_gather(mu_v, [ifull])
            rssp = plsc.load_gather(rstd_v, [ifull])
            for k in range(4):
                e = emb_v[i, pl.ds(k * L, L)]
                emb_v[i, pl.ds(k * L, L)] = (e - musp) * (rssp * gk[k]) + bk[k]

        pltpu.async_copy(emb_v, out_emb.at[pl.ds(tb, CH)], osem_b[b])
        pltpu.async_copy(unc_v, out_unc.at[pl.ds(tb, CH)], usem_b[b])

        @pl.when(c + 2 < NCHUNK)
        def _():
            pltpu.make_async_copy(ids.at[pl.ds(tb + 2 * CH, CH)], idx_b[b],
                                  isem_b[b]).wait()
            pltpu.async_copy(table.at[idx_b[b]], rows_b[b], gsem_b[b])

    def pair_body(p, carry):
        for b in (0, 1):
            run_chunk(b, 2 * p + b)
        return carry

    lax.fori_loop(0, NCHUNK // 2, pair_body, 0)

    for b in (0, 1):
        tl = t0 + (NCHUNK - 2 + b) * CH
        pltpu.make_async_copy(
            emb_b[b], out_emb.at[pl.ds(tl, CH)], osem_b[b]).wait()
        pltpu.make_async_copy(
            unc_b[b], out_unc.at[pl.ds(tl, CH)], usem_b[b]).wait()


@jax.jit
def _run(ids, table, sm_flat, pos_flat, tte0, ln_gamma, ln_beta):
    mesh = plsc.VectorSubcoreMesh(core_axis_name="c", subcore_axis_name="s",
                                  num_cores=NC, num_subcores=NSUB)
    kern = pl.kernel(
        _body,
        out_type=[
            jax.ShapeDtypeStruct((TOKENS, D), _F32),
            jax.ShapeDtypeStruct((TOKENS,), _F32),
        ],
        mesh=mesh,
        compiler_params=pltpu.CompilerParams(needs_layout_passes=False),
        scratch_types=[
            pltpu.VMEM((S * D,), _F32),
            pltpu.VMEM((16,), _F32),
            pltpu.VMEM((D,), _F32),
            pltpu.VMEM((D,), _F32),
            pltpu.VMEM((D,), _F32),
            pltpu.VMEM((CH,), _I32),
            pltpu.VMEM((CH,), _I32),
            pltpu.VMEM((CH, NSTATE * D), _F32),
            pltpu.VMEM((CH, NSTATE * D), _F32),
            pltpu.VMEM((CH, D), _F32),
            pltpu.VMEM((CH, D), _F32),
            pltpu.VMEM((CH,), _F32),
            pltpu.VMEM((CH,), _F32),
            pltpu.VMEM((CH,), _F32),
            pltpu.VMEM((CH,), _F32),
            pltpu.VMEM((CH,), _F32),
            pltpu.VMEM((CH,), _F32),
            pltpu.VMEM((CH,), _F32),
        ] + [pltpu.SemaphoreType.DMA] * 8,
    )
    return kern(ids, table, sm_flat, pos_flat, tte0, ln_gamma, ln_beta)


def kernel(input_ids, state_embeddings, superposition_matrix,
           position_embeddings, token_type_embeddings, ln_gamma, ln_beta):
    ids = input_ids.reshape(-1)
    table = state_embeddings.reshape(VOCAB, NSTATE * D)
    sm_flat = superposition_matrix.reshape(-1)
    pos_flat = position_embeddings[:S].reshape(-1)
    tte0 = token_type_embeddings[0]
    emb, unc = _run(ids, table, sm_flat, pos_flat, tte0, ln_gamma, ln_beta)
    return emb.reshape(B, S, D), unc.reshape(B, S)
